# baseline (speedup 1.0000x reference)
import functools

import jax
import jax.numpy as jnp
import numpy as np
from jax.experimental import pallas as pl

B, T, H, W = 2, 2, 64, 256
FOV_UP = 3.0 * np.pi / 180.0
FOV_DOWN = -25.0 * np.pi / 180.0
MASK_THRESHOLD = 0.5
BT = B * T
N = H * W + 1
NPAD = 16896
TQ = 512
TR = 4224
NQT = NPAD // TQ
NRT = NPAD // TR
NPROB = 2 * BT


def _chamfer_body(qn_ref, qn2_ref, rt_ref, rt2_ref, s_ref, c_ref):
    q = pl.program_id(1)
    q1 = qn_ref[0]
    q2 = qn2_ref[0]
    ax = q1[:, 0:1] + q1[:, 3:4]
    ay = q1[:, 1:2] + q1[:, 4:5]
    az = q1[:, 2:3] + q1[:, 5:6]
    na = ax * ax + ay * ay + az * az
    dn = (((1,), (0,)), ((), ()))
    m = jnp.full((TQ, 1), jnp.inf, jnp.float32)
    for t in range(NRT):
        r1 = rt_ref[0, :, pl.ds(t * TR, TR)]
        r2 = rt2_ref[0, :, pl.ds(t * TR, TR)]
        rx = r1[0:1] + r2[0:1]
        ry = r1[1:2] + r2[1:2]
        rz = r1[2:3] + r2[2:3]
        nb = 0.25 * (rx * rx + ry * ry + rz * rz)
        v = (jax.lax.dot_general(q1, r1, dn, preferred_element_type=jnp.float32)
             + jax.lax.dot_general(q2, r2, dn, preferred_element_type=jnp.float32))
        v = v + nb
        m = jnp.minimum(m, jnp.min(v, axis=1, keepdims=True))
    dist = na + m
    s = jnp.sum(dist)
    c = jnp.sum((dist > 0.0).astype(jnp.float32))
    sv = jnp.full((1, 1, 128), s, jnp.float32)
    cv = jnp.full((1, 1, 128), c, jnp.float32)

    @pl.when(q == 0)
    def _():
        s_ref[...] = sv
        c_ref[...] = cv

    @pl.when(q != 0)
    def _():
        s_ref[...] = s_ref[...] + sv
        c_ref[...] = c_ref[...] + cv


@functools.partial(jax.jit)
def _chamfer(output_rv, output_mask_logits, target):
    mask_prob = jax.nn.sigmoid(output_mask_logits)
    masked_rv = jnp.where(mask_prob > MASK_THRESHOLD, output_rv, -1.0)
    rv = masked_rv.reshape(BT, H, W)

    h = jnp.arange(H, dtype=jnp.float32)
    w = jnp.arange(W, dtype=jnp.float32)
    yaw = -((w + 0.5) / W * 2.0 - 1.0) * jnp.pi
    pitch = (1.0 - (h + 0.5) / H) * (FOV_UP - FOV_DOWN) + FOV_DOWN
    yaw2 = jnp.broadcast_to(yaw[None, :], (H, W))
    pitch2 = jnp.broadcast_to(pitch[:, None], (H, W))
    x = rv * (jnp.cos(pitch2) * jnp.cos(yaw2))[None]
    y = rv * (jnp.cos(pitch2) * jnp.sin(yaw2))[None]
    z = rv * jnp.sin(pitch2)[None]
    valid = rv > 0.0
    ox = jnp.where(valid, x, 1000.0).reshape(BT, H * W)
    oy = jnp.where(valid, y, 1000.0).reshape(BT, H * W)
    oz = jnp.where(valid, z, 1000.0).reshape(BT, H * W)

    tvalid = (target[:, :, 0] >= 0.0).reshape(BT, H * W)
    tx = jnp.where(tvalid, target[:, :, 1].reshape(BT, H * W), 1000.0)
    ty = jnp.where(tvalid, target[:, :, 2].reshape(BT, H * W), 1000.0)
    tz = jnp.where(tvalid, target[:, :, 3].reshape(BT, H * W), 1000.0)

    def split(v):
        hi = v.astype(jnp.bfloat16).astype(jnp.float32)
        lo = (v - hi).astype(jnp.bfloat16).astype(jnp.float32)
        return hi, lo

    def build(cx, cy, cz):
        cx = jnp.pad(cx, ((0, 0), (0, NPAD - H * W)), constant_values=1000.0)
        cy = jnp.pad(cy, ((0, 0), (0, NPAD - H * W)), constant_values=1000.0)
        cz = jnp.pad(cz, ((0, 0), (0, NPAD - H * W)), constant_values=1000.0)
        xh, xl = split(cx)
        yh, yl = split(cy)
        zh, zl = split(cz)
        zero = jnp.zeros_like(cx)
        a1 = jnp.stack([xh, yh, zh, xl, yl, zl, zero, zero], axis=-1)
        a2 = jnp.stack([xh, yh, zh, zero, zero, zero, zero, zero], axis=-1)
        b1 = jnp.stack([-2 * xh, -2 * yh, -2 * zh, -2 * xh, -2 * yh, -2 * zh,
                        zero, zero], axis=1)
        b2 = jnp.stack([-2 * xl, -2 * yl, -2 * zl, zero, zero, zero,
                        zero, zero], axis=1)
        return a1, a2, b1, b2

    ao1, ao2, bo1, bo2 = build(ox, oy, oz)
    at1, at2, bt1, bt2 = build(tx, ty, tz)

    qn1 = jnp.concatenate([ao1, at1], axis=0)
    qn2 = jnp.concatenate([ao2, at2], axis=0)
    rt1 = jnp.concatenate([bt1, bo1], axis=0)
    rt2 = jnp.concatenate([bt2, bo2], axis=0)

    s, c = pl.pallas_call(
        _chamfer_body,
        grid=(NPROB, NQT),
        in_specs=[
            pl.BlockSpec((1, TQ, 8), lambda p, q: (p, q, 0)),
            pl.BlockSpec((1, TQ, 8), lambda p, q: (p, q, 0)),
            pl.BlockSpec((1, 8, NPAD), lambda p, q: (p, 0, 0)),
            pl.BlockSpec((1, 8, NPAD), lambda p, q: (p, 0, 0)),
        ],
        out_specs=[
            pl.BlockSpec((1, 1, 128), lambda p, q: (p, 0, 0)),
            pl.BlockSpec((1, 1, 128), lambda p, q: (p, 0, 0)),
        ],
        out_shape=[
            jax.ShapeDtypeStruct((NPROB, 1, 128), jnp.float32),
            jax.ShapeDtypeStruct((NPROB, 1, 128), jnp.float32),
        ],
    )(qn1, qn2, rt1, rt2)
    s = s[:, 0, 0]
    c = c[:, 0, 0]

    dist_combined = s[:BT] / c[:BT] + s[BT:] / c[BT:]
    chamfer_distances_tensor = dist_combined.reshape(T, B)
    chamf_dist_t = jnp.mean(chamfer_distances_tensor, axis=1)
    return chamf_dist_t, chamfer_distances_tensor


def kernel(output_rv, output_mask_logits, target):
    return _chamfer(output_rv, output_mask_logits, target)
